# baseline (speedup 1.0000x reference)
import functools

import jax
import jax.numpy as jnp
from jax import lax
from jax.experimental import pallas as pl
from jax.experimental.pallas import tpu as pltpu
from jax.experimental.pallas import tpu_sc as plsc

N = 16384
H = 8
D = 64
R = 3
K = 8
NH = 3
BS = 128
HD = D + R
HP = 128
G = NH * H
NIDX = G * N

B_PREP = 512
B_ATTN = 512
B_FIN = 256
W_SC = 128


def _prep_body(x_ref, coords_ref, wq_ref, wk_ref, wv_ref, rpe_ref, alpha_ref,
               ln1s_ref, ln1b_ref, q_ref, k_ref, v_ref, qh_ref, kh_ref):
    x = x_ref[...]
    m = jnp.mean(x, axis=-1, keepdims=True)
    v = jnp.mean((x - m) ** 2, axis=-1, keepdims=True)
    xn = (x - m) / jnp.sqrt(v + 1e-5) * ln1s_ref[...][None, :] + ln1b_ref[...][None, :]

    dn = (((1,), (1,)), ((), ()))
    q = lax.dot_general(xn, wq_ref[...], dn, preferred_element_type=jnp.float32)
    k = lax.dot_general(xn, wk_ref[...], dn, preferred_element_type=jnp.float32)
    vv = lax.dot_general(xn, wv_ref[...], dn, preferred_element_type=jnp.float32)

    coords = coords_ref[...]
    rw = rpe_ref[...]
    sel = (lax.broadcasted_iota(jnp.int32, (R * K, R), 0) // K
           == lax.broadcasted_iota(jnp.int32, (R * K, R), 1)).astype(jnp.float32)
    zeros_pad = jnp.zeros((x.shape[0], HP - HD), jnp.float32)
    zeros_pad_v = jnp.zeros((x.shape[0], HP - D), jnp.float32)
    for h in range(H):
        s_h = jnp.sum(rw[h * D:(h + 1) * D, :], axis=0, keepdims=True)
        e_h = jnp.exp(jnp.minimum(s_h, 50.0))
        qw = lax.dot_general(e_h, sel, (((1,), (0,)), ((), ())),
                             preferred_element_type=jnp.float32)
        srw = jnp.sqrt(2.0 * qw) * coords
        q_h = q[:, h * D:(h + 1) * D]
        k_h = k[:, h * D:(h + 1) * D]
        q_ref[h] = jnp.concatenate([q_h, srw, zeros_pad], axis=-1)
        k_ref[h] = jnp.concatenate([k_h, srw, zeros_pad], axis=-1)
        v_ref[h] = jnp.concatenate([vv[:, h * D:(h + 1) * D], zeros_pad_v], axis=-1)
        a_main = alpha_ref[h, :D, :]
        a_coord = alpha_ref[h, D:, :]
        dt = (((0,), (1,)), ((), ()))
        qh_ref[h] = (lax.dot_general(a_main, q_h, dt,
                                     preferred_element_type=jnp.float32)
                     + lax.dot_general(a_coord, srw, dt,
                                       preferred_element_type=jnp.float32))
        kh_ref[h] = (lax.dot_general(a_main, k_h, dt,
                                     preferred_element_type=jnp.float32)
                     + lax.dot_general(a_coord, srw, dt,
                                       preferred_element_type=jnp.float32))


def _prep_call(x, coords, wq, wk, wv, rpe_w, alpha, ln1_s, ln1_b):
    nb = N // B_PREP
    full = lambda shp: pl.BlockSpec(shp, lambda i: tuple(0 for _ in shp))
    return pl.pallas_call(
        _prep_body,
        grid=(nb,),
        in_specs=[
            pl.BlockSpec((B_PREP, D), lambda i: (i, 0)),
            pl.BlockSpec((B_PREP, R), lambda i: (i, 0)),
            full((H * D, D)), full((H * D, D)), full((H * D, D)),
            full((H * D, R * K)), full((H, HD, NH)),
            full((D,)), full((D,)),
        ],
        out_specs=[
            pl.BlockSpec((H, B_PREP, HP), lambda i: (0, i, 0)),
            pl.BlockSpec((H, B_PREP, HP), lambda i: (0, i, 0)),
            pl.BlockSpec((H, B_PREP, HP), lambda i: (0, i, 0)),
            pl.BlockSpec((H, NH, B_PREP), lambda i: (0, 0, i)),
            pl.BlockSpec((H, NH, B_PREP), lambda i: (0, 0, i)),
        ],
        out_shape=[
            jax.ShapeDtypeStruct((H, N, HP), jnp.float32),
            jax.ShapeDtypeStruct((H, N, HP), jnp.float32),
            jax.ShapeDtypeStruct((H, N, HP), jnp.float32),
            jax.ShapeDtypeStruct((H, NH, N), jnp.float32),
            jax.ShapeDtypeStruct((H, NH, N), jnp.float32),
        ],
    )(x, coords, wq, wk, wv, rpe_w, alpha, ln1_s, ln1_b)


NSORT = 2 * G
NWORK = 32
CHUNK = N // 16


def _sc_argsort(qh2, kh2, shifts_f):
    mesh = plsc.VectorSubcoreMesh(core_axis_name="c", subcore_axis_name="s")
    import dataclasses
    cp = pltpu.CompilerParams()
    if "needs_layout_passes" in pltpu.CompilerParams.__dataclass_fields__:
        cp = dataclasses.replace(cp, needs_layout_passes=False)

    @functools.partial(
        pl.kernel, mesh=mesh,
        out_type=[
            jax.ShapeDtypeStruct((G, N), jnp.int32),
            jax.ShapeDtypeStruct((G, N), jnp.int32),
            jax.ShapeDtypeStruct((G, N), jnp.int32),
        ],
        scratch_types=[
            pltpu.VMEM((N,), jnp.int32),
            pltpu.VMEM((N,), jnp.int32),
            pltpu.VMEM((N,), jnp.int32),
            pltpu.VMEM((N,), jnp.int32),
            pltpu.VMEM((N,), jnp.float32),
            pltpu.VMEM((N,), jnp.float32),
            pltpu.VMEM((N,), jnp.float32),
            pltpu.VMEM((4096,), jnp.int32),
            pltpu.VMEM((16,), jnp.float32),
            pltpu.VMEM((16,), jnp.float32),
            pltpu.SMEM((4,), jnp.int32),
        ],
        compiler_params=cp,
    )
    def kern(qh_hbm, kh_hbm, sh_hbm, qi_hbm, ki_hbm, si_hbm,
             k0, k1, v0, v1, hp, ho, shv, hist, mxv, mnv, run):
        wid = lax.axis_index("s") * 2 + lax.axis_index("c")
        lane = lax.broadcasted_iota(jnp.int32, (16,), 0)
        lane_chunk = lane * CHUNK
        ones = jnp.ones((16,), jnp.int32)
        zeros = jnp.zeros((16,), jnp.int32)
        pltpu.sync_copy(sh_hbm, shv)

        for a0 in range(2):
            a = wid + NWORK * a0

            @pl.when(a < NSORT)
            def _():
                is_q = a < G
                am = lax.rem(a, G)
                h = am // NH
                rowoff = h * N
                nhoff = lax.rem(am, NH) * (H * N)

                @pl.when(is_q)
                def _():
                    pltpu.sync_copy(qh_hbm.at[am], hp)
                    pltpu.sync_copy(kh_hbm.at[am], ho)

                @pl.when(jnp.logical_not(is_q))
                def _():
                    pltpu.sync_copy(kh_hbm.at[am], hp)
                    pltpu.sync_copy(qh_hbm.at[am], ho)

                mxv[...] = jnp.full((16,), -jnp.inf, jnp.float32)
                mnv[...] = jnp.full((16,), jnp.inf, jnp.float32)

                @pl.loop(0, CHUNK)
                def _(i):
                    s = pl.ds(i * 16, 16)
                    pv = hp[s]
                    ov = ho[s]
                    mxv[...] = jnp.maximum(mxv[...], jnp.maximum(pv, ov))
                    mnv[...] = jnp.minimum(mnv[...], jnp.minimum(pv, ov))

                rng = jnp.max(mxv[...]) - jnp.min(mnv[...])

                @pl.loop(0, CHUNK)
                def _(i):
                    s = pl.ds(i * 16, 16)
                    kv = hp[s] + shv[s] * rng
                    b = plsc.bitcast(kv, jnp.int32)
                    k0[s] = jnp.where(
                        b < 0, ~b, b | jnp.int32(-2147483648))
                    v0[s] = lane + (i * 16)

                for p in range(4):
                    src_k, src_v = (k0, v0) if p % 2 == 0 else (k1, v1)
                    dst_k, dst_v = (k1, v1) if p % 2 == 0 else (k0, v0)
                    shift = 8 * p

                    @pl.loop(0, 256)
                    def _(b):
                        hist[pl.ds(b * 16, 16)] = zeros

                    @pl.loop(0, CHUNK)
                    def _(i):
                        idx = lane_chunk + i
                        kv = plsc.load_gather(src_k, [idx])
                        digit = lax.shift_right_logical(kv, shift) & 255
                        plsc.addupdate_scatter(hist, [digit * 16 + lane], ones)

                    run[0] = 0

                    @pl.loop(0, 256)
                    def _(b):
                        h_b = hist[pl.ds(b * 16, 16)]
                        excl = plsc.cumsum(h_b) - h_b
                        r = run[0]
                        hist[pl.ds(b * 16, 16)] = excl + r
                        run[0] = r + jnp.sum(h_b)

                    @pl.loop(0, CHUNK)
                    def _(i):
                        idx = lane_chunk + i
                        kv = plsc.load_gather(src_k, [idx])
                        vv = plsc.load_gather(src_v, [idx])
                        digit = lax.shift_right_logical(kv, shift) & 255
                        dl = digit * 16 + lane
                        cur = plsc.load_gather(hist, [dl])
                        plsc.store_scatter(dst_k, [cur], kv)
                        plsc.store_scatter(dst_v, [cur], vv)
                        plsc.addupdate_scatter(hist, [dl], ones)

                @pl.loop(0, CHUNK)
                def _(i):
                    s = pl.ds(i * 16, 16)
                    v0[s] = v0[s] + rowoff

                @pl.when(is_q)
                def _():
                    pltpu.sync_copy(v0, qi_hbm.at[am])

                    @pl.loop(0, CHUNK)
                    def _(i):
                        s = pl.ds(i * 16, 16)
                        k1[s] = v0[s] + nhoff

                    pltpu.sync_copy(k1, si_hbm.at[am])

                @pl.when(jnp.logical_not(is_q))
                def _():
                    pltpu.sync_copy(v0, ki_hbm.at[am])

    return kern(qh2, kh2, shifts_f)


def _sc_gather(q_tab, k_tab, v_tab, q_idx, k_idx):
    mesh = plsc.VectorSubcoreMesh(core_axis_name="c", subcore_axis_name="s")

    @functools.partial(
        pl.kernel, mesh=mesh,
        out_type=[
            jax.ShapeDtypeStruct((NIDX, HP), jnp.float32),
            jax.ShapeDtypeStruct((NIDX, HP), jnp.float32),
            jax.ShapeDtypeStruct((NIDX, HP), jnp.float32),
        ],
    )
    def kern(qt_hbm, kt_hbm, vt_hbm, qi_hbm, ki_hbm, oq_hbm, ok_hbm, ov_hbm):
        def body(qi_v, ki_v, oq_v, ok_v, ov_v):
            pltpu.sync_copy(qt_hbm.at[qi_v.at[0]], oq_v)
            pltpu.sync_copy(kt_hbm.at[ki_v.at[0]], ok_v)
            pltpu.sync_copy(vt_hbm.at[ki_v.at[0]], ov_v)

        pw = N // W_SC
        pltpu.emit_pipeline(
            body,
            grid=(NIDX // W_SC,),
            in_specs=[
                pl.BlockSpec((1, W_SC), lambda i: (i // pw, i % pw)),
                pl.BlockSpec((1, W_SC), lambda i: (i // pw, i % pw)),
            ],
            out_specs=[
                pl.BlockSpec((W_SC, HP), lambda i: (i, 0)),
                pl.BlockSpec((W_SC, HP), lambda i: (i, 0)),
                pl.BlockSpec((W_SC, HP), lambda i: (i, 0)),
            ],
            core_axis_name=("c", "s"),
            dimension_semantics=(pltpu.PARALLEL,),
        )(qi_hbm, ki_hbm, oq_hbm, ok_hbm, ov_hbm)

    return kern(q_tab, k_tab, v_tab, q_idx, k_idx)


def _sc_scatter(rows, idx):
    mesh = plsc.VectorSubcoreMesh(core_axis_name="c", subcore_axis_name="s")

    @functools.partial(
        pl.kernel, mesh=mesh,
        out_type=jax.ShapeDtypeStruct((NH * H * N, HP), jnp.float32),
    )
    def kern(rows_hbm, idx_hbm, out_hbm):
        def body(rows_v, idx_v):
            pltpu.sync_copy(rows_v, out_hbm.at[idx_v.at[0]])

        pw = N // W_SC
        pltpu.emit_pipeline(
            body,
            grid=(NIDX // W_SC,),
            in_specs=[
                pl.BlockSpec((W_SC, HP), lambda i: (i, 0)),
                pl.BlockSpec((1, W_SC), lambda i: (i // pw, i % pw)),
            ],
            out_specs=[],
            core_axis_name=("c", "s"),
            dimension_semantics=(pltpu.PARALLEL,),
        )(rows_hbm, idx_hbm)

    return kern(rows, idx)


def _attn_body(sq_ref, sk_ref, sv_ref, out_ref):
    nb = B_ATTN // BS
    excol = (lax.broadcasted_iota(jnp.int32, (BS, HP), 1) == D
             ).astype(jnp.float32)
    for b in range(nb):
        sq = sq_ref[0, b * BS:(b + 1) * BS, :]
        sk = sk_ref[0, b * BS:(b + 1) * BS, :]
        sv = sv_ref[0, b * BS:(b + 1) * BS, :]
        qsq = -0.5 * jnp.sum(sq * sq, axis=-1, keepdims=True)
        ksq = -0.5 * jnp.sum(sk * sk, axis=-1)
        scores = lax.dot_general(sq, sk, (((1,), (1,)), ((), ())),
                                 preferred_element_type=jnp.float32)
        dists = jnp.exp(jnp.minimum(scores + qsq + ksq[None, :], 0.0))
        o = lax.dot_general(dists, sv + excol, (((1,), (0,)), ((), ())),
                            preferred_element_type=jnp.float32)
        out_ref[0, b * BS:(b + 1) * BS, :] = o + 1e-20 * excol


def _attn_call(sq, sk, sv):
    nb = N // B_ATTN
    return pl.pallas_call(
        _attn_body,
        grid=(G, nb),
        in_specs=[
            pl.BlockSpec((1, B_ATTN, HP), lambda g, i: (g, i, 0)),
            pl.BlockSpec((1, B_ATTN, HP), lambda g, i: (g, i, 0)),
            pl.BlockSpec((1, B_ATTN, HP), lambda g, i: (g, i, 0)),
        ],
        out_specs=pl.BlockSpec((1, B_ATTN, HP), lambda g, i: (g, i, 0)),
        out_shape=jax.ShapeDtypeStruct((G, N, HP), jnp.float32),
    )(sq, sk, sv)


def _final_body(so_ref, x_ref, outw_ref, outb_ref, ln2s_ref, ln2b_ref,
                ff1w_ref, ff1b_ref, ff2w_ref, ff2b_ref, y_ref):
    s = so_ref[...]
    s = s[0] + s[1] + s[2]
    o = s[..., :D]
    den = s[..., D:D + 1]
    outh = o / den
    dn = (((1,), (1,)), ((), ()))
    aggr = outb_ref[...][None, :]
    for h in range(H):
        w_h = outw_ref[:, h * D:(h + 1) * D]
        aggr = aggr + lax.dot_general(outh[h], w_h, dn,
                                      preferred_element_type=jnp.float32)
    x1 = x_ref[...] + aggr
    m = jnp.mean(x1, axis=-1, keepdims=True)
    v = jnp.mean((x1 - m) ** 2, axis=-1, keepdims=True)
    xn2 = (x1 - m) / jnp.sqrt(v + 1e-5) * ln2s_ref[...][None, :] + ln2b_ref[...][None, :]
    h1 = lax.dot_general(xn2, ff1w_ref[...], dn,
                         preferred_element_type=jnp.float32) + ff1b_ref[...][None, :]
    h1 = h1 * jax.nn.sigmoid(h1)
    h2 = lax.dot_general(h1, ff2w_ref[...], dn,
                         preferred_element_type=jnp.float32) + ff2b_ref[...][None, :]
    y_ref[...] = x1 + h2


def _final_call(so, x, out_w, out_b, ln2_s, ln2_b, ff1_w, ff1_b, ff2_w, ff2_b):
    nb = N // B_FIN
    full = lambda shp: pl.BlockSpec(shp, lambda i: tuple(0 for _ in shp))
    return pl.pallas_call(
        _final_body,
        grid=(nb,),
        in_specs=[
            pl.BlockSpec((NH, H, B_FIN, HP), lambda i: (0, 0, i, 0)),
            pl.BlockSpec((B_FIN, D), lambda i: (i, 0)),
            full((D, H * D)), full((D,)), full((D,)), full((D,)),
            full((D, D)), full((D,)), full((D, D)), full((D,)),
        ],
        out_specs=pl.BlockSpec((B_FIN, D), lambda i: (i, 0)),
        out_shape=jax.ShapeDtypeStruct((N, D), jnp.float32),
    )(so, x, out_w, out_b, ln2_s, ln2_b, ff1_w, ff1_b, ff2_w, ff2_b)


def kernel(x, coords, combined_shifts, wq, wk, wv, out_w, out_b, rpe_w,
           ln1_s, ln1_b, ln2_s, ln2_b, ff1_w, ff1_b, ff2_w, ff2_b, alpha):
    q_tab, k_tab, v_tab, qh, kh = _prep_call(
        x, coords, wq, wk, wv, rpe_w, alpha, ln1_s, ln1_b)

    q_flat, k_flat, s_idx = _sc_argsort(
        qh.reshape(G, N), kh.reshape(G, N),
        combined_shifts.astype(jnp.float32))

    sq, sk, sv = _sc_gather(q_tab.reshape(H * N, HP), k_tab.reshape(H * N, HP),
                            v_tab.reshape(H * N, HP), q_flat, k_flat)

    so = _attn_call(sq.reshape(G, N, HP), sk.reshape(G, N, HP),
                    sv.reshape(G, N, HP))

    uns = _sc_scatter(so.reshape(NIDX, HP), s_idx)

    return _final_call(uns.reshape(NH, H, N, HP), x, out_w, out_b,
                       ln2_s, ln2_b, ff1_w, ff1_b, ff2_w, ff2_b)


# speedup vs baseline: 1.0379x; 1.0379x over previous
import functools

import jax
import jax.numpy as jnp
from jax import lax
from jax.experimental import pallas as pl
from jax.experimental.pallas import tpu as pltpu
from jax.experimental.pallas import tpu_sc as plsc

N = 16384
H = 8
D = 64
R = 3
K = 8
NH = 3
BS = 128
HD = D + R
HP = 128
G = NH * H
NIDX = G * N

B_PREP = 512
B_ATTN = 512
B_FIN = 256
W_SC = 128


def _prep_body(x_ref, coords_ref, wq_ref, wk_ref, wv_ref, rpe_ref, alpha_ref,
               ln1s_ref, ln1b_ref, q_ref, k_ref, v_ref, qh_ref, kh_ref):
    x = x_ref[...]
    m = jnp.mean(x, axis=-1, keepdims=True)
    v = jnp.mean((x - m) ** 2, axis=-1, keepdims=True)
    xn = (x - m) / jnp.sqrt(v + 1e-5) * ln1s_ref[...][None, :] + ln1b_ref[...][None, :]

    dn = (((1,), (1,)), ((), ()))
    q = lax.dot_general(xn, wq_ref[...], dn, preferred_element_type=jnp.float32)
    k = lax.dot_general(xn, wk_ref[...], dn, preferred_element_type=jnp.float32)
    vv = lax.dot_general(xn, wv_ref[...], dn, preferred_element_type=jnp.float32)

    coords = coords_ref[...]
    rw = rpe_ref[...]
    sel = (lax.broadcasted_iota(jnp.int32, (R * K, R), 0) // K
           == lax.broadcasted_iota(jnp.int32, (R * K, R), 1)).astype(jnp.float32)
    zeros_pad = jnp.zeros((x.shape[0], HP - HD), jnp.float32)
    zeros_pad_v = jnp.zeros((x.shape[0], HP - D), jnp.float32)
    for h in range(H):
        s_h = jnp.sum(rw[h * D:(h + 1) * D, :], axis=0, keepdims=True)
        e_h = jnp.exp(jnp.minimum(s_h, 50.0))
        qw = lax.dot_general(e_h, sel, (((1,), (0,)), ((), ())),
                             preferred_element_type=jnp.float32)
        srw = jnp.sqrt(2.0 * qw) * coords
        q_h = q[:, h * D:(h + 1) * D]
        k_h = k[:, h * D:(h + 1) * D]
        q_ref[h] = jnp.concatenate([q_h, srw, zeros_pad], axis=-1)
        k_ref[h] = jnp.concatenate([k_h, srw, zeros_pad], axis=-1)
        v_ref[h] = jnp.concatenate([vv[:, h * D:(h + 1) * D], zeros_pad_v], axis=-1)
        a_main = alpha_ref[h, :D, :]
        a_coord = alpha_ref[h, D:, :]
        dt = (((0,), (1,)), ((), ()))
        qh_ref[h] = (lax.dot_general(a_main, q_h, dt,
                                     preferred_element_type=jnp.float32)
                     + lax.dot_general(a_coord, srw, dt,
                                       preferred_element_type=jnp.float32))
        kh_ref[h] = (lax.dot_general(a_main, k_h, dt,
                                     preferred_element_type=jnp.float32)
                     + lax.dot_general(a_coord, srw, dt,
                                       preferred_element_type=jnp.float32))


def _prep_call(x, coords, wq, wk, wv, rpe_w, alpha, ln1_s, ln1_b):
    nb = N // B_PREP
    full = lambda shp: pl.BlockSpec(shp, lambda i: tuple(0 for _ in shp))
    return pl.pallas_call(
        _prep_body,
        grid=(nb,),
        in_specs=[
            pl.BlockSpec((B_PREP, D), lambda i: (i, 0)),
            pl.BlockSpec((B_PREP, R), lambda i: (i, 0)),
            full((H * D, D)), full((H * D, D)), full((H * D, D)),
            full((H * D, R * K)), full((H, HD, NH)),
            full((D,)), full((D,)),
        ],
        out_specs=[
            pl.BlockSpec((H, B_PREP, HP), lambda i: (0, i, 0)),
            pl.BlockSpec((H, B_PREP, HP), lambda i: (0, i, 0)),
            pl.BlockSpec((H, B_PREP, HP), lambda i: (0, i, 0)),
            pl.BlockSpec((H, NH, B_PREP), lambda i: (0, 0, i)),
            pl.BlockSpec((H, NH, B_PREP), lambda i: (0, 0, i)),
        ],
        out_shape=[
            jax.ShapeDtypeStruct((H, N, HP), jnp.float32),
            jax.ShapeDtypeStruct((H, N, HP), jnp.float32),
            jax.ShapeDtypeStruct((H, N, HP), jnp.float32),
            jax.ShapeDtypeStruct((H, NH, N), jnp.float32),
            jax.ShapeDtypeStruct((H, NH, N), jnp.float32),
        ],
    )(x, coords, wq, wk, wv, rpe_w, alpha, ln1_s, ln1_b)


NSORT = 2 * G
NWORK = 32
CHUNK = N // 16


def _sc_argsort(qh2, kh2, shifts_f):
    mesh = plsc.VectorSubcoreMesh(core_axis_name="c", subcore_axis_name="s")
    import dataclasses
    cp = pltpu.CompilerParams()
    if "needs_layout_passes" in pltpu.CompilerParams.__dataclass_fields__:
        cp = dataclasses.replace(cp, needs_layout_passes=False)

    @functools.partial(
        pl.kernel, mesh=mesh,
        out_type=[
            jax.ShapeDtypeStruct((G, N), jnp.int32),
            jax.ShapeDtypeStruct((G, N), jnp.int32),
            jax.ShapeDtypeStruct((G, N), jnp.int32),
        ],
        scratch_types=[
            pltpu.VMEM((N,), jnp.int32),
            pltpu.VMEM((N,), jnp.int32),
            pltpu.VMEM((N,), jnp.int32),
            pltpu.VMEM((N,), jnp.int32),
            pltpu.VMEM((N,), jnp.float32),
            pltpu.VMEM((N,), jnp.float32),
            pltpu.VMEM((N,), jnp.float32),
            pltpu.VMEM((4096,), jnp.int32),
            pltpu.VMEM((16,), jnp.float32),
            pltpu.VMEM((16,), jnp.float32),
            pltpu.SMEM((4,), jnp.int32),
        ],
        compiler_params=cp,
    )
    def kern(qh_hbm, kh_hbm, sh_hbm, qi_hbm, ki_hbm, si_hbm,
             k0, k1, v0, v1, hp, ho, shv, hist, mxv, mnv, run):
        wid = lax.axis_index("s") * 2 + lax.axis_index("c")
        lane = lax.broadcasted_iota(jnp.int32, (16,), 0)
        lane_chunk = lane * CHUNK
        ones = jnp.ones((16,), jnp.int32)
        zeros = jnp.zeros((16,), jnp.int32)
        pltpu.sync_copy(sh_hbm, shv)

        for a0 in range(2):
            a = wid + NWORK * a0

            @pl.when(a < NSORT)
            def _():
                is_q = a < G
                am = lax.rem(a, G)
                h = am // NH
                rowoff = h * N
                nhoff = lax.rem(am, NH) * (H * N)

                @pl.when(is_q)
                def _():
                    pltpu.sync_copy(qh_hbm.at[am], hp)
                    pltpu.sync_copy(kh_hbm.at[am], ho)

                @pl.when(jnp.logical_not(is_q))
                def _():
                    pltpu.sync_copy(kh_hbm.at[am], hp)
                    pltpu.sync_copy(qh_hbm.at[am], ho)

                mxv[...] = jnp.full((16,), -jnp.inf, jnp.float32)
                mnv[...] = jnp.full((16,), jnp.inf, jnp.float32)

                @pl.loop(0, CHUNK)
                def _(i):
                    s = pl.ds(i * 16, 16)
                    pv = hp[s]
                    ov = ho[s]
                    mxv[...] = jnp.maximum(mxv[...], jnp.maximum(pv, ov))
                    mnv[...] = jnp.minimum(mnv[...], jnp.minimum(pv, ov))

                lo = jnp.min(mnv[...])
                rng = jnp.max(mxv[...]) - lo
                rng_v = jnp.zeros((16,), jnp.float32) + rng
                scale = jnp.where(rng_v > 0.0, 16777215.0 / (16.0 * rng_v), 0.0)

                @pl.loop(0, CHUNK)
                def _(i):
                    s = pl.ds(i * 16, 16)
                    kv = hp[s] + shv[s] * rng
                    k0[s] = jnp.minimum((kv - lo) * scale, 16777215.0).astype(jnp.int32)
                    v0[s] = lane + (i * 16)

                for p in range(3):
                    src_k, src_v = (k0, v0) if p % 2 == 0 else (k1, v1)
                    dst_k, dst_v = (k1, v1) if p % 2 == 0 else (k0, v0)
                    shift = 8 * p

                    @pl.loop(0, 256)
                    def _(b):
                        hist[pl.ds(b * 16, 16)] = zeros

                    @pl.loop(0, CHUNK)
                    def _(i):
                        idx = lane_chunk + i
                        kv = plsc.load_gather(src_k, [idx])
                        digit = lax.shift_right_logical(kv, shift) & 255
                        plsc.addupdate_scatter(hist, [digit * 16 + lane], ones)

                    run[0] = 0

                    @pl.loop(0, 256)
                    def _(b):
                        h_b = hist[pl.ds(b * 16, 16)]
                        excl = plsc.cumsum(h_b) - h_b
                        r = run[0]
                        hist[pl.ds(b * 16, 16)] = excl + r
                        run[0] = r + jnp.sum(h_b)

                    @pl.loop(0, CHUNK)
                    def _(i):
                        idx = lane_chunk + i
                        kv = plsc.load_gather(src_k, [idx])
                        vv = plsc.load_gather(src_v, [idx])
                        digit = lax.shift_right_logical(kv, shift) & 255
                        dl = digit * 16 + lane
                        cur = plsc.load_gather(hist, [dl])
                        plsc.store_scatter(dst_k, [cur], kv)
                        plsc.store_scatter(dst_v, [cur], vv)
                        plsc.addupdate_scatter(hist, [dl], ones)

                @pl.loop(0, CHUNK)
                def _(i):
                    s = pl.ds(i * 16, 16)
                    v1[s] = v1[s] + rowoff

                @pl.when(is_q)
                def _():
                    pltpu.sync_copy(v1, qi_hbm.at[am])

                    @pl.loop(0, CHUNK)
                    def _(i):
                        s = pl.ds(i * 16, 16)
                        k0[s] = v1[s] + nhoff

                    pltpu.sync_copy(k0, si_hbm.at[am])

                @pl.when(jnp.logical_not(is_q))
                def _():
                    pltpu.sync_copy(v1, ki_hbm.at[am])

    return kern(qh2, kh2, shifts_f)


def _sc_gather(q_tab, k_tab, v_tab, q_idx, k_idx):
    mesh = plsc.VectorSubcoreMesh(core_axis_name="c", subcore_axis_name="s")

    @functools.partial(
        pl.kernel, mesh=mesh,
        out_type=[
            jax.ShapeDtypeStruct((NIDX, HP), jnp.float32),
            jax.ShapeDtypeStruct((NIDX, HP), jnp.float32),
            jax.ShapeDtypeStruct((NIDX, HP), jnp.float32),
        ],
    )
    def kern(qt_hbm, kt_hbm, vt_hbm, qi_hbm, ki_hbm, oq_hbm, ok_hbm, ov_hbm):
        def body(qi_v, ki_v, oq_v, ok_v, ov_v):
            pltpu.sync_copy(qt_hbm.at[qi_v.at[0]], oq_v)
            pltpu.sync_copy(kt_hbm.at[ki_v.at[0]], ok_v)
            pltpu.sync_copy(vt_hbm.at[ki_v.at[0]], ov_v)

        pw = N // W_SC
        pltpu.emit_pipeline(
            body,
            grid=(NIDX // W_SC,),
            in_specs=[
                pl.BlockSpec((1, W_SC), lambda i: (i // pw, i % pw)),
                pl.BlockSpec((1, W_SC), lambda i: (i // pw, i % pw)),
            ],
            out_specs=[
                pl.BlockSpec((W_SC, HP), lambda i: (i, 0)),
                pl.BlockSpec((W_SC, HP), lambda i: (i, 0)),
                pl.BlockSpec((W_SC, HP), lambda i: (i, 0)),
            ],
            core_axis_name=("c", "s"),
            dimension_semantics=(pltpu.PARALLEL,),
        )(qi_hbm, ki_hbm, oq_hbm, ok_hbm, ov_hbm)

    return kern(q_tab, k_tab, v_tab, q_idx, k_idx)


def _sc_scatter(rows, idx):
    mesh = plsc.VectorSubcoreMesh(core_axis_name="c", subcore_axis_name="s")

    @functools.partial(
        pl.kernel, mesh=mesh,
        out_type=jax.ShapeDtypeStruct((NH * H * N, HP), jnp.float32),
    )
    def kern(rows_hbm, idx_hbm, out_hbm):
        def body(rows_v, idx_v):
            pltpu.sync_copy(rows_v, out_hbm.at[idx_v.at[0]])

        pw = N // W_SC
        pltpu.emit_pipeline(
            body,
            grid=(NIDX // W_SC,),
            in_specs=[
                pl.BlockSpec((W_SC, HP), lambda i: (i, 0)),
                pl.BlockSpec((1, W_SC), lambda i: (i // pw, i % pw)),
            ],
            out_specs=[],
            core_axis_name=("c", "s"),
            dimension_semantics=(pltpu.PARALLEL,),
        )(rows_hbm, idx_hbm)

    return kern(rows, idx)


def _attn_body(sq_ref, sk_ref, sv_ref, out_ref):
    nb = B_ATTN // BS
    excol = (lax.broadcasted_iota(jnp.int32, (BS, HP), 1) == D
             ).astype(jnp.float32)
    for b in range(nb):
        sq = sq_ref[0, b * BS:(b + 1) * BS, :]
        sk = sk_ref[0, b * BS:(b + 1) * BS, :]
        sv = sv_ref[0, b * BS:(b + 1) * BS, :]
        qsq = -0.5 * jnp.sum(sq * sq, axis=-1, keepdims=True)
        ksq = -0.5 * jnp.sum(sk * sk, axis=-1)
        scores = lax.dot_general(sq, sk, (((1,), (1,)), ((), ())),
                                 preferred_element_type=jnp.float32)
        dists = jnp.exp(jnp.minimum(scores + qsq + ksq[None, :], 0.0))
        o = lax.dot_general(dists, sv + excol, (((1,), (0,)), ((), ())),
                            preferred_element_type=jnp.float32)
        out_ref[0, b * BS:(b + 1) * BS, :] = o + 1e-20 * excol


def _attn_call(sq, sk, sv):
    nb = N // B_ATTN
    return pl.pallas_call(
        _attn_body,
        grid=(G, nb),
        in_specs=[
            pl.BlockSpec((1, B_ATTN, HP), lambda g, i: (g, i, 0)),
            pl.BlockSpec((1, B_ATTN, HP), lambda g, i: (g, i, 0)),
            pl.BlockSpec((1, B_ATTN, HP), lambda g, i: (g, i, 0)),
        ],
        out_specs=pl.BlockSpec((1, B_ATTN, HP), lambda g, i: (g, i, 0)),
        out_shape=jax.ShapeDtypeStruct((G, N, HP), jnp.float32),
    )(sq, sk, sv)


def _final_body(so_ref, x_ref, outw_ref, outb_ref, ln2s_ref, ln2b_ref,
                ff1w_ref, ff1b_ref, ff2w_ref, ff2b_ref, y_ref):
    s = so_ref[...]
    s = s[0] + s[1] + s[2]
    o = s[..., :D]
    den = s[..., D:D + 1]
    outh = o / den
    dn = (((1,), (1,)), ((), ()))
    aggr = outb_ref[...][None, :]
    for h in range(H):
        w_h = outw_ref[:, h * D:(h + 1) * D]
        aggr = aggr + lax.dot_general(outh[h], w_h, dn,
                                      preferred_element_type=jnp.float32)
    x1 = x_ref[...] + aggr
    m = jnp.mean(x1, axis=-1, keepdims=True)
    v = jnp.mean((x1 - m) ** 2, axis=-1, keepdims=True)
    xn2 = (x1 - m) / jnp.sqrt(v + 1e-5) * ln2s_ref[...][None, :] + ln2b_ref[...][None, :]
    h1 = lax.dot_general(xn2, ff1w_ref[...], dn,
                         preferred_element_type=jnp.float32) + ff1b_ref[...][None, :]
    h1 = h1 * jax.nn.sigmoid(h1)
    h2 = lax.dot_general(h1, ff2w_ref[...], dn,
                         preferred_element_type=jnp.float32) + ff2b_ref[...][None, :]
    y_ref[...] = x1 + h2


def _final_call(so, x, out_w, out_b, ln2_s, ln2_b, ff1_w, ff1_b, ff2_w, ff2_b):
    nb = N // B_FIN
    full = lambda shp: pl.BlockSpec(shp, lambda i: tuple(0 for _ in shp))
    return pl.pallas_call(
        _final_body,
        grid=(nb,),
        in_specs=[
            pl.BlockSpec((NH, H, B_FIN, HP), lambda i: (0, 0, i, 0)),
            pl.BlockSpec((B_FIN, D), lambda i: (i, 0)),
            full((D, H * D)), full((D,)), full((D,)), full((D,)),
            full((D, D)), full((D,)), full((D, D)), full((D,)),
        ],
        out_specs=pl.BlockSpec((B_FIN, D), lambda i: (i, 0)),
        out_shape=jax.ShapeDtypeStruct((N, D), jnp.float32),
    )(so, x, out_w, out_b, ln2_s, ln2_b, ff1_w, ff1_b, ff2_w, ff2_b)


def kernel(x, coords, combined_shifts, wq, wk, wv, out_w, out_b, rpe_w,
           ln1_s, ln1_b, ln2_s, ln2_b, ff1_w, ff1_b, ff2_w, ff2_b, alpha):
    q_tab, k_tab, v_tab, qh, kh = _prep_call(
        x, coords, wq, wk, wv, rpe_w, alpha, ln1_s, ln1_b)

    q_flat, k_flat, s_idx = _sc_argsort(
        qh.reshape(G, N), kh.reshape(G, N),
        combined_shifts.astype(jnp.float32))

    sq, sk, sv = _sc_gather(q_tab.reshape(H * N, HP), k_tab.reshape(H * N, HP),
                            v_tab.reshape(H * N, HP), q_flat, k_flat)

    so = _attn_call(sq.reshape(G, N, HP), sk.reshape(G, N, HP),
                    sv.reshape(G, N, HP))

    uns = _sc_scatter(so.reshape(NIDX, HP), s_idx)

    return _final_call(uns.reshape(NH, H, N, HP), x, out_w, out_b,
                       ln2_s, ln2_b, ff1_w, ff1_b, ff2_w, ff2_b)


# speedup vs baseline: 1.0769x; 1.0376x over previous
import functools

import jax
import jax.numpy as jnp
from jax import lax
from jax.experimental import pallas as pl
from jax.experimental.pallas import tpu as pltpu
from jax.experimental.pallas import tpu_sc as plsc

N = 16384
H = 8
D = 64
R = 3
K = 8
NH = 3
BS = 128
HD = D + R
HP = 128
G = NH * H
NIDX = G * N

B_PREP = 512
B_ATTN = 512
B_FIN = 256
W_SC = 128


def _prep_body(x_ref, coords_ref, wq_ref, wk_ref, wv_ref, rpe_ref, alpha_ref,
               ln1s_ref, ln1b_ref, q_ref, k_ref, v_ref, qh_ref, kh_ref):
    x = x_ref[...]
    m = jnp.mean(x, axis=-1, keepdims=True)
    v = jnp.mean((x - m) ** 2, axis=-1, keepdims=True)
    xn = (x - m) / jnp.sqrt(v + 1e-5) * ln1s_ref[...][None, :] + ln1b_ref[...][None, :]

    dn = (((1,), (1,)), ((), ()))
    q = lax.dot_general(xn, wq_ref[...], dn, preferred_element_type=jnp.float32)
    k = lax.dot_general(xn, wk_ref[...], dn, preferred_element_type=jnp.float32)
    vv = lax.dot_general(xn, wv_ref[...], dn, preferred_element_type=jnp.float32)

    coords = coords_ref[...]
    rw = rpe_ref[...]
    sel = (lax.broadcasted_iota(jnp.int32, (R * K, R), 0) // K
           == lax.broadcasted_iota(jnp.int32, (R * K, R), 1)).astype(jnp.float32)
    zeros_pad = jnp.zeros((x.shape[0], HP - HD), jnp.float32)
    zeros_pad_v = jnp.zeros((x.shape[0], HP - D), jnp.float32)
    for h in range(H):
        s_h = jnp.sum(rw[h * D:(h + 1) * D, :], axis=0, keepdims=True)
        e_h = jnp.exp(jnp.minimum(s_h, 50.0))
        qw = lax.dot_general(e_h, sel, (((1,), (0,)), ((), ())),
                             preferred_element_type=jnp.float32)
        srw = jnp.sqrt(2.0 * qw) * coords
        q_h = q[:, h * D:(h + 1) * D]
        k_h = k[:, h * D:(h + 1) * D]
        q_ref[h] = jnp.concatenate([q_h, srw, zeros_pad], axis=-1)
        k_ref[h] = jnp.concatenate([k_h, srw, zeros_pad], axis=-1)
        v_ref[h] = jnp.concatenate([vv[:, h * D:(h + 1) * D], zeros_pad_v], axis=-1)
        a_main = alpha_ref[h, :D, :]
        a_coord = alpha_ref[h, D:, :]
        dt = (((0,), (1,)), ((), ()))
        qh_ref[h] = (lax.dot_general(a_main, q_h, dt,
                                     preferred_element_type=jnp.float32)
                     + lax.dot_general(a_coord, srw, dt,
                                       preferred_element_type=jnp.float32))
        kh_ref[h] = (lax.dot_general(a_main, k_h, dt,
                                     preferred_element_type=jnp.float32)
                     + lax.dot_general(a_coord, srw, dt,
                                       preferred_element_type=jnp.float32))


def _prep_call(x, coords, wq, wk, wv, rpe_w, alpha, ln1_s, ln1_b):
    nb = N // B_PREP
    full = lambda shp: pl.BlockSpec(shp, lambda i: tuple(0 for _ in shp))
    return pl.pallas_call(
        _prep_body,
        grid=(nb,),
        in_specs=[
            pl.BlockSpec((B_PREP, D), lambda i: (i, 0)),
            pl.BlockSpec((B_PREP, R), lambda i: (i, 0)),
            full((H * D, D)), full((H * D, D)), full((H * D, D)),
            full((H * D, R * K)), full((H, HD, NH)),
            full((D,)), full((D,)),
        ],
        out_specs=[
            pl.BlockSpec((H, B_PREP, HP), lambda i: (0, i, 0)),
            pl.BlockSpec((H, B_PREP, HP), lambda i: (0, i, 0)),
            pl.BlockSpec((H, B_PREP, HP), lambda i: (0, i, 0)),
            pl.BlockSpec((H, NH, B_PREP), lambda i: (0, 0, i)),
            pl.BlockSpec((H, NH, B_PREP), lambda i: (0, 0, i)),
        ],
        out_shape=[
            jax.ShapeDtypeStruct((H, N, HP), jnp.float32),
            jax.ShapeDtypeStruct((H, N, HP), jnp.float32),
            jax.ShapeDtypeStruct((H, N, HP), jnp.float32),
            jax.ShapeDtypeStruct((H, NH, N), jnp.float32),
            jax.ShapeDtypeStruct((H, NH, N), jnp.float32),
        ],
    )(x, coords, wq, wk, wv, rpe_w, alpha, ln1_s, ln1_b)


NSORT = 2 * G
NWORK = 32
CHUNK = N // 16


def _sc_argsort(qh2, kh2, shifts_f):
    mesh = plsc.VectorSubcoreMesh(core_axis_name="c", subcore_axis_name="s")
    import dataclasses
    cp = pltpu.CompilerParams()
    if "needs_layout_passes" in pltpu.CompilerParams.__dataclass_fields__:
        cp = dataclasses.replace(cp, needs_layout_passes=False)

    @functools.partial(
        pl.kernel, mesh=mesh,
        out_type=[
            jax.ShapeDtypeStruct((G, N), jnp.int32),
            jax.ShapeDtypeStruct((G, N), jnp.int32),
            jax.ShapeDtypeStruct((G, N), jnp.int32),
        ],
        scratch_types=[
            pltpu.VMEM((N,), jnp.int32),
            pltpu.VMEM((N,), jnp.int32),
            pltpu.VMEM((N,), jnp.int32),
            pltpu.VMEM((N,), jnp.int32),
            pltpu.VMEM((N,), jnp.float32),
            pltpu.VMEM((N,), jnp.float32),
            pltpu.VMEM((N,), jnp.float32),
            pltpu.VMEM((4096,), jnp.int32),
            pltpu.VMEM((16,), jnp.float32),
            pltpu.VMEM((16,), jnp.float32),
            pltpu.SMEM((4,), jnp.int32),
        ],
        compiler_params=cp,
    )
    def kern(qh_hbm, kh_hbm, sh_hbm, qi_hbm, ki_hbm, si_hbm,
             k0, k1, v0, v1, hp, ho, shv, hist, mxv, mnv, run):
        wid = lax.axis_index("s") * 2 + lax.axis_index("c")
        lane = lax.broadcasted_iota(jnp.int32, (16,), 0)
        lane_chunk = lane * CHUNK
        ones = jnp.ones((16,), jnp.int32)
        zeros = jnp.zeros((16,), jnp.int32)
        pltpu.sync_copy(sh_hbm, shv)

        for a0 in range(2):
            a = wid + NWORK * a0

            @pl.when(a < NSORT)
            def _():
                is_q = a < G
                am = lax.rem(a, G)
                h = am // NH
                rowoff = h * N
                nhoff = lax.rem(am, NH) * (H * N)

                @pl.when(is_q)
                def _():
                    pltpu.sync_copy(qh_hbm.at[am], hp)
                    pltpu.sync_copy(kh_hbm.at[am], ho)

                @pl.when(jnp.logical_not(is_q))
                def _():
                    pltpu.sync_copy(kh_hbm.at[am], hp)
                    pltpu.sync_copy(qh_hbm.at[am], ho)

                mxv[...] = jnp.full((16,), -jnp.inf, jnp.float32)
                mnv[...] = jnp.full((16,), jnp.inf, jnp.float32)

                @pl.loop(0, CHUNK)
                def _(i):
                    s = pl.ds(i * 16, 16)
                    pv = hp[s]
                    ov = ho[s]
                    mxv[...] = jnp.maximum(mxv[...], jnp.maximum(pv, ov))
                    mnv[...] = jnp.minimum(mnv[...], jnp.minimum(pv, ov))

                lo = jnp.min(mnv[...])
                rng = jnp.max(mxv[...]) - lo
                rng_v = jnp.zeros((16,), jnp.float32) + rng
                scale = jnp.where(rng_v > 0.0, 16777215.0 / (16.0 * rng_v), 0.0)

                @pl.loop(0, CHUNK)
                def _(i):
                    s = pl.ds(i * 16, 16)
                    kv = hp[s] + shv[s] * rng
                    k0[s] = jnp.minimum((kv - lo) * scale, 16777215.0).astype(jnp.int32)
                    v0[s] = lane + (i * 16)

                for p in range(3):
                    src_k, src_v = (k0, v0) if p % 2 == 0 else (k1, v1)
                    dst_k, dst_v = (k1, v1) if p % 2 == 0 else (k0, v0)
                    shift = 8 * p

                    @pl.loop(0, 256)
                    def _(b):
                        hist[pl.ds(b * 16, 16)] = zeros

                    @pl.loop(0, CHUNK)
                    def _(i):
                        idx = lane_chunk + i
                        kv = plsc.load_gather(src_k, [idx])
                        digit = lax.shift_right_logical(kv, shift) & 255
                        plsc.addupdate_scatter(hist, [digit * 16 + lane], ones)

                    run[0] = 0

                    @pl.loop(0, 256)
                    def _(b):
                        h_b = hist[pl.ds(b * 16, 16)]
                        excl = plsc.cumsum(h_b) - h_b
                        r = run[0]
                        hist[pl.ds(b * 16, 16)] = excl + r
                        run[0] = r + jnp.sum(h_b)

                    @pl.loop(0, CHUNK)
                    def _(i):
                        idx = lane_chunk + i
                        kv = plsc.load_gather(src_k, [idx])
                        vv = plsc.load_gather(src_v, [idx])
                        digit = lax.shift_right_logical(kv, shift) & 255
                        dl = digit * 16 + lane
                        cur = plsc.load_gather(hist, [dl])
                        plsc.store_scatter(dst_k, [cur], kv)
                        plsc.store_scatter(dst_v, [cur], vv)
                        plsc.addupdate_scatter(hist, [dl], ones)

                @pl.loop(0, CHUNK)
                def _(i):
                    s = pl.ds(i * 16, 16)
                    v1[s] = v1[s] + rowoff

                @pl.when(is_q)
                def _():
                    pltpu.sync_copy(v1, qi_hbm.at[am])

                    @pl.loop(0, CHUNK)
                    def _(i):
                        s = pl.ds(i * 16, 16)
                        k0[s] = v1[s] + nhoff

                    pltpu.sync_copy(k0, si_hbm.at[am])

                @pl.when(jnp.logical_not(is_q))
                def _():
                    pltpu.sync_copy(v1, ki_hbm.at[am])

    return kern(qh2, kh2, shifts_f)


def _sc_gather(q_tab, k_tab, v_tab, q_idx, k_idx):
    mesh = plsc.VectorSubcoreMesh(core_axis_name="c", subcore_axis_name="s")

    @functools.partial(
        pl.kernel, mesh=mesh,
        out_type=[
            jax.ShapeDtypeStruct((NIDX, HP), jnp.float32),
            jax.ShapeDtypeStruct((NIDX, HP), jnp.float32),
            jax.ShapeDtypeStruct((NIDX, HP), jnp.float32),
        ],
    )
    def kern(qt_hbm, kt_hbm, vt_hbm, qi_hbm, ki_hbm, oq_hbm, ok_hbm, ov_hbm):
        def body(qi_v, ki_v, oq_v, ok_v, ov_v):
            def inner(s1, s2, s3):
                c1 = pltpu.async_copy(qt_hbm.at[qi_v.at[0]], oq_v, s1)
                c2 = pltpu.async_copy(kt_hbm.at[ki_v.at[0]], ok_v, s2)
                c3 = pltpu.async_copy(vt_hbm.at[ki_v.at[0]], ov_v, s3)
                c1.wait()
                c2.wait()
                c3.wait()

            pl.run_scoped(inner, pltpu.SemaphoreType.DMA,
                          pltpu.SemaphoreType.DMA, pltpu.SemaphoreType.DMA)

        pw = N // W_SC
        pltpu.emit_pipeline(
            body,
            grid=(NIDX // W_SC,),
            in_specs=[
                pl.BlockSpec((1, W_SC), lambda i: (i // pw, i % pw)),
                pl.BlockSpec((1, W_SC), lambda i: (i // pw, i % pw)),
            ],
            out_specs=[
                pl.BlockSpec((W_SC, HP), lambda i: (i, 0)),
                pl.BlockSpec((W_SC, HP), lambda i: (i, 0)),
                pl.BlockSpec((W_SC, HP), lambda i: (i, 0)),
            ],
            core_axis_name=("c", "s"),
            dimension_semantics=(pltpu.PARALLEL,),
        )(qi_hbm, ki_hbm, oq_hbm, ok_hbm, ov_hbm)

    return kern(q_tab, k_tab, v_tab, q_idx, k_idx)


def _sc_scatter(rows, idx):
    mesh = plsc.VectorSubcoreMesh(core_axis_name="c", subcore_axis_name="s")

    @functools.partial(
        pl.kernel, mesh=mesh,
        out_type=jax.ShapeDtypeStruct((NH * H * N, HP), jnp.float32),
    )
    def kern(rows_hbm, idx_hbm, out_hbm):
        def body(rows_v, idx_v):
            pltpu.sync_copy(rows_v, out_hbm.at[idx_v.at[0]])

        pw = N // W_SC
        pltpu.emit_pipeline(
            body,
            grid=(NIDX // W_SC,),
            in_specs=[
                pl.BlockSpec((W_SC, HP), lambda i: (i, 0)),
                pl.BlockSpec((1, W_SC), lambda i: (i // pw, i % pw)),
            ],
            out_specs=[],
            core_axis_name=("c", "s"),
            dimension_semantics=(pltpu.PARALLEL,),
        )(rows_hbm, idx_hbm)

    return kern(rows, idx)


def _attn_body(sq_ref, sk_ref, sv_ref, out_ref):
    nb = B_ATTN // BS
    excol = (lax.broadcasted_iota(jnp.int32, (BS, HP), 1) == D
             ).astype(jnp.float32)
    for b in range(nb):
        sq = sq_ref[0, b * BS:(b + 1) * BS, :]
        sk = sk_ref[0, b * BS:(b + 1) * BS, :]
        sv = sv_ref[0, b * BS:(b + 1) * BS, :]
        qsq = -0.5 * jnp.sum(sq * sq, axis=-1, keepdims=True)
        ksq = -0.5 * jnp.sum(sk * sk, axis=-1)
        scores = lax.dot_general(sq, sk, (((1,), (1,)), ((), ())),
                                 preferred_element_type=jnp.float32)
        dists = jnp.exp(jnp.minimum(scores + qsq + ksq[None, :], 0.0))
        o = lax.dot_general(dists, sv + excol, (((1,), (0,)), ((), ())),
                            preferred_element_type=jnp.float32)
        out_ref[0, b * BS:(b + 1) * BS, :] = o + 1e-20 * excol


def _attn_call(sq, sk, sv):
    nb = N // B_ATTN
    return pl.pallas_call(
        _attn_body,
        grid=(G, nb),
        in_specs=[
            pl.BlockSpec((1, B_ATTN, HP), lambda g, i: (g, i, 0)),
            pl.BlockSpec((1, B_ATTN, HP), lambda g, i: (g, i, 0)),
            pl.BlockSpec((1, B_ATTN, HP), lambda g, i: (g, i, 0)),
        ],
        out_specs=pl.BlockSpec((1, B_ATTN, HP), lambda g, i: (g, i, 0)),
        out_shape=jax.ShapeDtypeStruct((G, N, HP), jnp.float32),
    )(sq, sk, sv)


def _final_body(so_ref, x_ref, outw_ref, outb_ref, ln2s_ref, ln2b_ref,
                ff1w_ref, ff1b_ref, ff2w_ref, ff2b_ref, y_ref):
    s = so_ref[...]
    s = s[0] + s[1] + s[2]
    o = s[..., :D]
    den = s[..., D:D + 1]
    outh = o / den
    dn = (((1,), (1,)), ((), ()))
    aggr = outb_ref[...][None, :]
    for h in range(H):
        w_h = outw_ref[:, h * D:(h + 1) * D]
        aggr = aggr + lax.dot_general(outh[h], w_h, dn,
                                      preferred_element_type=jnp.float32)
    x1 = x_ref[...] + aggr
    m = jnp.mean(x1, axis=-1, keepdims=True)
    v = jnp.mean((x1 - m) ** 2, axis=-1, keepdims=True)
    xn2 = (x1 - m) / jnp.sqrt(v + 1e-5) * ln2s_ref[...][None, :] + ln2b_ref[...][None, :]
    h1 = lax.dot_general(xn2, ff1w_ref[...], dn,
                         preferred_element_type=jnp.float32) + ff1b_ref[...][None, :]
    h1 = h1 * jax.nn.sigmoid(h1)
    h2 = lax.dot_general(h1, ff2w_ref[...], dn,
                         preferred_element_type=jnp.float32) + ff2b_ref[...][None, :]
    y_ref[...] = x1 + h2


def _final_call(so, x, out_w, out_b, ln2_s, ln2_b, ff1_w, ff1_b, ff2_w, ff2_b):
    nb = N // B_FIN
    full = lambda shp: pl.BlockSpec(shp, lambda i: tuple(0 for _ in shp))
    return pl.pallas_call(
        _final_body,
        grid=(nb,),
        in_specs=[
            pl.BlockSpec((NH, H, B_FIN, HP), lambda i: (0, 0, i, 0)),
            pl.BlockSpec((B_FIN, D), lambda i: (i, 0)),
            full((D, H * D)), full((D,)), full((D,)), full((D,)),
            full((D, D)), full((D,)), full((D, D)), full((D,)),
        ],
        out_specs=pl.BlockSpec((B_FIN, D), lambda i: (i, 0)),
        out_shape=jax.ShapeDtypeStruct((N, D), jnp.float32),
    )(so, x, out_w, out_b, ln2_s, ln2_b, ff1_w, ff1_b, ff2_w, ff2_b)


def kernel(x, coords, combined_shifts, wq, wk, wv, out_w, out_b, rpe_w,
           ln1_s, ln1_b, ln2_s, ln2_b, ff1_w, ff1_b, ff2_w, ff2_b, alpha):
    q_tab, k_tab, v_tab, qh, kh = _prep_call(
        x, coords, wq, wk, wv, rpe_w, alpha, ln1_s, ln1_b)

    q_flat, k_flat, s_idx = _sc_argsort(
        qh.reshape(G, N), kh.reshape(G, N),
        combined_shifts.astype(jnp.float32))

    sq, sk, sv = _sc_gather(q_tab.reshape(H * N, HP), k_tab.reshape(H * N, HP),
                            v_tab.reshape(H * N, HP), q_flat, k_flat)

    so = _attn_call(sq.reshape(G, N, HP), sk.reshape(G, N, HP),
                    sv.reshape(G, N, HP))

    uns = _sc_scatter(so.reshape(NIDX, HP), s_idx)

    return _final_call(uns.reshape(NH, H, N, HP), x, out_w, out_b,
                       ln2_s, ln2_b, ff1_w, ff1_b, ff2_w, ff2_b)


# speedup vs baseline: 1.2547x; 1.1651x over previous
import functools

import jax
import jax.numpy as jnp
from jax import lax
from jax.experimental import pallas as pl
from jax.experimental.pallas import tpu as pltpu
from jax.experimental.pallas import tpu_sc as plsc

N = 16384
H = 8
D = 64
R = 3
K = 8
NH = 3
BS = 128
HD = D + R
HP = 128
G = NH * H
NIDX = G * N

B_PREP = 512
B_ATTN = 2048
B_FIN = 256
W_SC = 128


def _prep_body(x_ref, coords_ref, wq_ref, wk_ref, wv_ref, rpe_ref, alpha_ref,
               ln1s_ref, ln1b_ref, q_ref, k_ref, v_ref, qh_ref, kh_ref):
    x = x_ref[...]
    m = jnp.mean(x, axis=-1, keepdims=True)
    v = jnp.mean((x - m) ** 2, axis=-1, keepdims=True)
    xn = (x - m) / jnp.sqrt(v + 1e-5) * ln1s_ref[...][None, :] + ln1b_ref[...][None, :]

    dn = (((1,), (1,)), ((), ()))
    q = lax.dot_general(xn, wq_ref[...], dn, preferred_element_type=jnp.float32)
    k = lax.dot_general(xn, wk_ref[...], dn, preferred_element_type=jnp.float32)
    vv = lax.dot_general(xn, wv_ref[...], dn, preferred_element_type=jnp.float32)

    coords = coords_ref[...]
    rw = rpe_ref[...]
    sel = (lax.broadcasted_iota(jnp.int32, (R * K, R), 0) // K
           == lax.broadcasted_iota(jnp.int32, (R * K, R), 1)).astype(jnp.float32)
    zeros_pad = jnp.zeros((x.shape[0], HP - HD), jnp.float32)
    zeros_pad_v = jnp.zeros((x.shape[0], HP - D), jnp.float32)
    for h in range(H):
        s_h = jnp.sum(rw[h * D:(h + 1) * D, :], axis=0, keepdims=True)
        e_h = jnp.exp(jnp.minimum(s_h, 50.0))
        qw = lax.dot_general(e_h, sel, (((1,), (0,)), ((), ())),
                             preferred_element_type=jnp.float32)
        srw = jnp.sqrt(2.0 * qw) * coords
        q_h = q[:, h * D:(h + 1) * D]
        k_h = k[:, h * D:(h + 1) * D]
        q_ref[h] = jnp.concatenate([q_h, srw, zeros_pad], axis=-1)
        k_ref[h] = jnp.concatenate([k_h, srw, zeros_pad], axis=-1)
        v_ref[h] = jnp.concatenate([vv[:, h * D:(h + 1) * D], zeros_pad_v], axis=-1)
        a_main = alpha_ref[h, :D, :]
        a_coord = alpha_ref[h, D:, :]
        dt = (((0,), (1,)), ((), ()))
        qh_ref[h] = (lax.dot_general(a_main, q_h, dt,
                                     preferred_element_type=jnp.float32)
                     + lax.dot_general(a_coord, srw, dt,
                                       preferred_element_type=jnp.float32))
        kh_ref[h] = (lax.dot_general(a_main, k_h, dt,
                                     preferred_element_type=jnp.float32)
                     + lax.dot_general(a_coord, srw, dt,
                                       preferred_element_type=jnp.float32))


def _prep_call(x, coords, wq, wk, wv, rpe_w, alpha, ln1_s, ln1_b):
    nb = N // B_PREP
    full = lambda shp: pl.BlockSpec(shp, lambda i: tuple(0 for _ in shp))
    return pl.pallas_call(
        _prep_body,
        grid=(nb,),
        in_specs=[
            pl.BlockSpec((B_PREP, D), lambda i: (i, 0)),
            pl.BlockSpec((B_PREP, R), lambda i: (i, 0)),
            full((H * D, D)), full((H * D, D)), full((H * D, D)),
            full((H * D, R * K)), full((H, HD, NH)),
            full((D,)), full((D,)),
        ],
        out_specs=[
            pl.BlockSpec((H, B_PREP, HP), lambda i: (0, i, 0)),
            pl.BlockSpec((H, B_PREP, HP), lambda i: (0, i, 0)),
            pl.BlockSpec((H, B_PREP, HP), lambda i: (0, i, 0)),
            pl.BlockSpec((H, NH, B_PREP), lambda i: (0, 0, i)),
            pl.BlockSpec((H, NH, B_PREP), lambda i: (0, 0, i)),
        ],
        out_shape=[
            jax.ShapeDtypeStruct((H, N, HP), jnp.float32),
            jax.ShapeDtypeStruct((H, N, HP), jnp.float32),
            jax.ShapeDtypeStruct((H, N, HP), jnp.float32),
            jax.ShapeDtypeStruct((H, NH, N), jnp.float32),
            jax.ShapeDtypeStruct((H, NH, N), jnp.float32),
        ],
    )(x, coords, wq, wk, wv, rpe_w, alpha, ln1_s, ln1_b)


NSORT = 2 * G
NWORK = 32
CHUNK = N // 16


def _sc_argsort(qh2, kh2, shifts_f):
    mesh = plsc.VectorSubcoreMesh(core_axis_name="c", subcore_axis_name="s")
    import dataclasses
    cp = pltpu.CompilerParams()
    if "needs_layout_passes" in pltpu.CompilerParams.__dataclass_fields__:
        cp = dataclasses.replace(cp, needs_layout_passes=False)

    @functools.partial(
        pl.kernel, mesh=mesh,
        out_type=[
            jax.ShapeDtypeStruct((G, N), jnp.int32),
            jax.ShapeDtypeStruct((G, N), jnp.int32),
            jax.ShapeDtypeStruct((G, N), jnp.int32),
        ],
        scratch_types=[
            pltpu.VMEM((N,), jnp.int32),
            pltpu.VMEM((N,), jnp.int32),
            pltpu.VMEM((N,), jnp.int32),
            pltpu.VMEM((N,), jnp.int32),
            pltpu.VMEM((N,), jnp.float32),
            pltpu.VMEM((N,), jnp.float32),
            pltpu.VMEM((N,), jnp.float32),
            pltpu.VMEM((4096,), jnp.int32),
            pltpu.VMEM((16,), jnp.float32),
            pltpu.VMEM((16,), jnp.float32),
            pltpu.SMEM((4,), jnp.int32),
        ],
        compiler_params=cp,
    )
    def kern(qh_hbm, kh_hbm, sh_hbm, qi_hbm, ki_hbm, si_hbm,
             k0, k1, v0, v1, hp, ho, shv, hist, mxv, mnv, run):
        wid = lax.axis_index("s") * 2 + lax.axis_index("c")
        lane = lax.broadcasted_iota(jnp.int32, (16,), 0)
        lane_chunk = lane * CHUNK
        ones = jnp.ones((16,), jnp.int32)
        zeros = jnp.zeros((16,), jnp.int32)
        pltpu.sync_copy(sh_hbm, shv)

        for a0 in range(2):
            a = wid + NWORK * a0

            @pl.when(a < NSORT)
            def _():
                is_q = a < G
                am = lax.rem(a, G)
                h = am // NH
                rowoff = h * N
                nhoff = lax.rem(am, NH) * (H * N)

                @pl.when(is_q)
                def _():
                    pltpu.sync_copy(qh_hbm.at[am], hp)
                    pltpu.sync_copy(kh_hbm.at[am], ho)

                @pl.when(jnp.logical_not(is_q))
                def _():
                    pltpu.sync_copy(kh_hbm.at[am], hp)
                    pltpu.sync_copy(qh_hbm.at[am], ho)

                mxv[...] = jnp.full((16,), -jnp.inf, jnp.float32)
                mnv[...] = jnp.full((16,), jnp.inf, jnp.float32)

                @pl.loop(0, CHUNK)
                def _(i):
                    s = pl.ds(i * 16, 16)
                    pv = hp[s]
                    ov = ho[s]
                    mxv[...] = jnp.maximum(mxv[...], jnp.maximum(pv, ov))
                    mnv[...] = jnp.minimum(mnv[...], jnp.minimum(pv, ov))

                lo = jnp.min(mnv[...])
                rng = jnp.max(mxv[...]) - lo
                rng_v = jnp.zeros((16,), jnp.float32) + rng
                scale = jnp.where(rng_v > 0.0, 16777215.0 / (16.0 * rng_v), 0.0)

                @pl.loop(0, CHUNK)
                def _(i):
                    s = pl.ds(i * 16, 16)
                    kv = hp[s] + shv[s] * rng
                    k0[s] = jnp.minimum((kv - lo) * scale, 16777215.0).astype(jnp.int32)
                    v0[s] = lane + (i * 16)

                for p in range(3):
                    src_k, src_v = (k0, v0) if p % 2 == 0 else (k1, v1)
                    dst_k, dst_v = (k1, v1) if p % 2 == 0 else (k0, v0)
                    shift = 8 * p

                    @pl.loop(0, 256)
                    def _(b):
                        hist[pl.ds(b * 16, 16)] = zeros

                    @pl.loop(0, CHUNK)
                    def _(i):
                        idx = lane_chunk + i
                        kv = plsc.load_gather(src_k, [idx])
                        digit = lax.shift_right_logical(kv, shift) & 255
                        plsc.addupdate_scatter(hist, [digit * 16 + lane], ones)

                    run[0] = 0

                    @pl.loop(0, 256)
                    def _(b):
                        h_b = hist[pl.ds(b * 16, 16)]
                        excl = plsc.cumsum(h_b) - h_b
                        r = run[0]
                        hist[pl.ds(b * 16, 16)] = excl + r
                        run[0] = r + jnp.sum(h_b)

                    @pl.loop(0, CHUNK)
                    def _(i):
                        idx = lane_chunk + i
                        kv = plsc.load_gather(src_k, [idx])
                        vv = plsc.load_gather(src_v, [idx])
                        digit = lax.shift_right_logical(kv, shift) & 255
                        dl = digit * 16 + lane
                        cur = plsc.load_gather(hist, [dl])
                        plsc.store_scatter(dst_k, [cur], kv)
                        plsc.store_scatter(dst_v, [cur], vv)
                        plsc.addupdate_scatter(hist, [dl], ones)

                @pl.loop(0, CHUNK)
                def _(i):
                    s = pl.ds(i * 16, 16)
                    v1[s] = v1[s] + rowoff

                @pl.when(is_q)
                def _():
                    pltpu.sync_copy(v1, qi_hbm.at[am])

                    @pl.loop(0, CHUNK)
                    def _(i):
                        s = pl.ds(i * 16, 16)
                        k0[s] = v1[s] + nhoff

                    pltpu.sync_copy(k0, si_hbm.at[am])

                @pl.when(jnp.logical_not(is_q))
                def _():
                    pltpu.sync_copy(v1, ki_hbm.at[am])

    return kern(qh2, kh2, shifts_f)


def _sc_gather(q_tab, k_tab, v_tab, q_idx, k_idx):
    mesh = plsc.VectorSubcoreMesh(core_axis_name="c", subcore_axis_name="s")

    @functools.partial(
        pl.kernel, mesh=mesh,
        out_type=[
            jax.ShapeDtypeStruct((NIDX, HP), jnp.float32),
            jax.ShapeDtypeStruct((NIDX, HP), jnp.float32),
            jax.ShapeDtypeStruct((NIDX, HP), jnp.float32),
        ],
    )
    def kern(qt_hbm, kt_hbm, vt_hbm, qi_hbm, ki_hbm, oq_hbm, ok_hbm, ov_hbm):
        def body(qi_v, ki_v, oq_v, ok_v, ov_v):
            def inner(s1, s2, s3):
                c1 = pltpu.async_copy(qt_hbm.at[qi_v.at[0]], oq_v, s1)
                c2 = pltpu.async_copy(kt_hbm.at[ki_v.at[0]], ok_v, s2)
                c3 = pltpu.async_copy(vt_hbm.at[ki_v.at[0]], ov_v, s3)
                c1.wait()
                c2.wait()
                c3.wait()

            pl.run_scoped(inner, pltpu.SemaphoreType.DMA,
                          pltpu.SemaphoreType.DMA, pltpu.SemaphoreType.DMA)

        pw = N // W_SC
        pltpu.emit_pipeline(
            body,
            grid=(NIDX // W_SC,),
            in_specs=[
                pl.BlockSpec((1, W_SC), lambda i: (i // pw, i % pw)),
                pl.BlockSpec((1, W_SC), lambda i: (i // pw, i % pw)),
            ],
            out_specs=[
                pl.BlockSpec((W_SC, HP), lambda i: (i, 0)),
                pl.BlockSpec((W_SC, HP), lambda i: (i, 0)),
                pl.BlockSpec((W_SC, HP), lambda i: (i, 0)),
            ],
            core_axis_name=("c", "s"),
            dimension_semantics=(pltpu.PARALLEL,),
        )(qi_hbm, ki_hbm, oq_hbm, ok_hbm, ov_hbm)

    return kern(q_tab, k_tab, v_tab, q_idx, k_idx)


def _sc_scatter(rows, idx):
    mesh = plsc.VectorSubcoreMesh(core_axis_name="c", subcore_axis_name="s")

    @functools.partial(
        pl.kernel, mesh=mesh,
        out_type=jax.ShapeDtypeStruct((NH * H * N, HP), jnp.float32),
    )
    def kern(rows_hbm, idx_hbm, out_hbm):
        def body(rows_v, idx_v):
            pltpu.sync_copy(rows_v, out_hbm.at[idx_v.at[0]])

        pw = N // W_SC
        pltpu.emit_pipeline(
            body,
            grid=(NIDX // W_SC,),
            in_specs=[
                pl.BlockSpec((W_SC, HP), lambda i: (i, 0)),
                pl.BlockSpec((1, W_SC), lambda i: (i // pw, i % pw)),
            ],
            out_specs=[],
            core_axis_name=("c", "s"),
            dimension_semantics=(pltpu.PARALLEL,),
        )(rows_hbm, idx_hbm)

    return kern(rows, idx)


def _attn_body(sq_ref, sk_ref, sv_ref, out_ref):
    nb = B_ATTN // BS
    excol = (lax.broadcasted_iota(jnp.int32, (BS, HP), 1) == D
             ).astype(jnp.float32)
    for b in range(nb):
        sq = sq_ref[0, b * BS:(b + 1) * BS, :]
        sk = sk_ref[0, b * BS:(b + 1) * BS, :]
        sv = sv_ref[0, b * BS:(b + 1) * BS, :]
        qsq = -0.5 * jnp.sum(sq * sq, axis=-1, keepdims=True)
        ksq = -0.5 * jnp.sum(sk * sk, axis=-1)
        scores = lax.dot_general(sq, sk, (((1,), (1,)), ((), ())),
                                 preferred_element_type=jnp.float32)
        dists = jnp.exp(jnp.minimum(scores + qsq + ksq[None, :], 0.0))
        o = lax.dot_general(dists, sv + excol, (((1,), (0,)), ((), ())),
                            preferred_element_type=jnp.float32)
        out_ref[0, b * BS:(b + 1) * BS, :] = o + 1e-20 * excol


def _attn_call(sq, sk, sv):
    nb = N // B_ATTN
    return pl.pallas_call(
        _attn_body,
        grid=(G, nb),
        in_specs=[
            pl.BlockSpec((1, B_ATTN, HP), lambda g, i: (g, i, 0)),
            pl.BlockSpec((1, B_ATTN, HP), lambda g, i: (g, i, 0)),
            pl.BlockSpec((1, B_ATTN, HP), lambda g, i: (g, i, 0)),
        ],
        out_specs=pl.BlockSpec((1, B_ATTN, HP), lambda g, i: (g, i, 0)),
        out_shape=jax.ShapeDtypeStruct((G, N, HP), jnp.float32),
    )(sq, sk, sv)


def _final_body(so_ref, x_ref, outw_ref, outb_ref, ln2s_ref, ln2b_ref,
                ff1w_ref, ff1b_ref, ff2w_ref, ff2b_ref, y_ref):
    s = so_ref[...]
    s = s[0] + s[1] + s[2]
    o = s[..., :D]
    den = s[..., D:D + 1]
    outh = o / den
    dn = (((1,), (1,)), ((), ()))
    aggr = outb_ref[...][None, :]
    for h in range(H):
        w_h = outw_ref[:, h * D:(h + 1) * D]
        aggr = aggr + lax.dot_general(outh[h], w_h, dn,
                                      preferred_element_type=jnp.float32)
    x1 = x_ref[...] + aggr
    m = jnp.mean(x1, axis=-1, keepdims=True)
    v = jnp.mean((x1 - m) ** 2, axis=-1, keepdims=True)
    xn2 = (x1 - m) / jnp.sqrt(v + 1e-5) * ln2s_ref[...][None, :] + ln2b_ref[...][None, :]
    h1 = lax.dot_general(xn2, ff1w_ref[...], dn,
                         preferred_element_type=jnp.float32) + ff1b_ref[...][None, :]
    h1 = h1 * jax.nn.sigmoid(h1)
    h2 = lax.dot_general(h1, ff2w_ref[...], dn,
                         preferred_element_type=jnp.float32) + ff2b_ref[...][None, :]
    y_ref[...] = x1 + h2


def _final_call(so, x, out_w, out_b, ln2_s, ln2_b, ff1_w, ff1_b, ff2_w, ff2_b):
    nb = N // B_FIN
    full = lambda shp: pl.BlockSpec(shp, lambda i: tuple(0 for _ in shp))
    return pl.pallas_call(
        _final_body,
        grid=(nb,),
        in_specs=[
            pl.BlockSpec((NH, H, B_FIN, HP), lambda i: (0, 0, i, 0)),
            pl.BlockSpec((B_FIN, D), lambda i: (i, 0)),
            full((D, H * D)), full((D,)), full((D,)), full((D,)),
            full((D, D)), full((D,)), full((D, D)), full((D,)),
        ],
        out_specs=pl.BlockSpec((B_FIN, D), lambda i: (i, 0)),
        out_shape=jax.ShapeDtypeStruct((N, D), jnp.float32),
    )(so, x, out_w, out_b, ln2_s, ln2_b, ff1_w, ff1_b, ff2_w, ff2_b)


def kernel(x, coords, combined_shifts, wq, wk, wv, out_w, out_b, rpe_w,
           ln1_s, ln1_b, ln2_s, ln2_b, ff1_w, ff1_b, ff2_w, ff2_b, alpha):
    q_tab, k_tab, v_tab, qh, kh = _prep_call(
        x, coords, wq, wk, wv, rpe_w, alpha, ln1_s, ln1_b)

    q_flat, k_flat, s_idx = _sc_argsort(
        qh.reshape(G, N), kh.reshape(G, N),
        combined_shifts.astype(jnp.float32))

    sq, sk, sv = _sc_gather(q_tab.reshape(H * N, HP), k_tab.reshape(H * N, HP),
                            v_tab.reshape(H * N, HP), q_flat, k_flat)

    so = _attn_call(sq.reshape(G, N, HP), sk.reshape(G, N, HP),
                    sv.reshape(G, N, HP))

    uns = _sc_scatter(so.reshape(NIDX, HP), s_idx)

    return _final_call(uns.reshape(NH, H, N, HP), x, out_w, out_b,
                       ln2_s, ln2_b, ff1_w, ff1_b, ff2_w, ff2_b)


# speedup vs baseline: 1.2690x; 1.0114x over previous
import functools

import jax
import jax.numpy as jnp
from jax import lax
from jax.experimental import pallas as pl
from jax.experimental.pallas import tpu as pltpu
from jax.experimental.pallas import tpu_sc as plsc

N = 16384
H = 8
D = 64
R = 3
K = 8
NH = 3
BS = 128
HD = D + R
HP = 128
G = NH * H
NIDX = G * N

B_PREP = 512
B_ATTN = 4096
B_FIN = 256
W_SC = 128


def _prep_body(x_ref, coords_ref, wq_ref, wk_ref, wv_ref, rpe_ref, alpha_ref,
               ln1s_ref, ln1b_ref, q_ref, k_ref, v_ref, qh_ref, kh_ref):
    x = x_ref[...]
    m = jnp.mean(x, axis=-1, keepdims=True)
    v = jnp.mean((x - m) ** 2, axis=-1, keepdims=True)
    xn = (x - m) / jnp.sqrt(v + 1e-5) * ln1s_ref[...][None, :] + ln1b_ref[...][None, :]

    dn = (((1,), (1,)), ((), ()))
    q = lax.dot_general(xn, wq_ref[...], dn, preferred_element_type=jnp.float32)
    k = lax.dot_general(xn, wk_ref[...], dn, preferred_element_type=jnp.float32)
    vv = lax.dot_general(xn, wv_ref[...], dn, preferred_element_type=jnp.float32)

    coords = coords_ref[...]
    rw = rpe_ref[...]
    sel = (lax.broadcasted_iota(jnp.int32, (R * K, R), 0) // K
           == lax.broadcasted_iota(jnp.int32, (R * K, R), 1)).astype(jnp.float32)
    zeros_pad = jnp.zeros((x.shape[0], HP - HD), jnp.float32)
    zeros_pad_v = jnp.zeros((x.shape[0], HP - D), jnp.float32)
    for h in range(H):
        s_h = jnp.sum(rw[h * D:(h + 1) * D, :], axis=0, keepdims=True)
        e_h = jnp.exp(jnp.minimum(s_h, 50.0))
        qw = lax.dot_general(e_h, sel, (((1,), (0,)), ((), ())),
                             preferred_element_type=jnp.float32)
        srw = jnp.sqrt(2.0 * qw) * coords
        q_h = q[:, h * D:(h + 1) * D]
        k_h = k[:, h * D:(h + 1) * D]
        q_ref[h] = jnp.concatenate([q_h, srw, zeros_pad], axis=-1)
        k_ref[h] = jnp.concatenate([k_h, srw, zeros_pad], axis=-1)
        v_ref[h] = jnp.concatenate([vv[:, h * D:(h + 1) * D], zeros_pad_v], axis=-1)
        a_main = alpha_ref[h, :D, :]
        a_coord = alpha_ref[h, D:, :]
        dt = (((0,), (1,)), ((), ()))
        qh_ref[h] = (lax.dot_general(a_main, q_h, dt,
                                     preferred_element_type=jnp.float32)
                     + lax.dot_general(a_coord, srw, dt,
                                       preferred_element_type=jnp.float32))
        kh_ref[h] = (lax.dot_general(a_main, k_h, dt,
                                     preferred_element_type=jnp.float32)
                     + lax.dot_general(a_coord, srw, dt,
                                       preferred_element_type=jnp.float32))


def _prep_call(x, coords, wq, wk, wv, rpe_w, alpha, ln1_s, ln1_b):
    nb = N // B_PREP
    full = lambda shp: pl.BlockSpec(shp, lambda i: tuple(0 for _ in shp))
    return pl.pallas_call(
        _prep_body,
        grid=(nb,),
        in_specs=[
            pl.BlockSpec((B_PREP, D), lambda i: (i, 0)),
            pl.BlockSpec((B_PREP, R), lambda i: (i, 0)),
            full((H * D, D)), full((H * D, D)), full((H * D, D)),
            full((H * D, R * K)), full((H, HD, NH)),
            full((D,)), full((D,)),
        ],
        out_specs=[
            pl.BlockSpec((H, B_PREP, HP), lambda i: (0, i, 0)),
            pl.BlockSpec((H, B_PREP, HP), lambda i: (0, i, 0)),
            pl.BlockSpec((H, B_PREP, HP), lambda i: (0, i, 0)),
            pl.BlockSpec((H, NH, B_PREP), lambda i: (0, 0, i)),
            pl.BlockSpec((H, NH, B_PREP), lambda i: (0, 0, i)),
        ],
        out_shape=[
            jax.ShapeDtypeStruct((H, N, HP), jnp.float32),
            jax.ShapeDtypeStruct((H, N, HP), jnp.float32),
            jax.ShapeDtypeStruct((H, N, HP), jnp.float32),
            jax.ShapeDtypeStruct((H, NH, N), jnp.float32),
            jax.ShapeDtypeStruct((H, NH, N), jnp.float32),
        ],
    )(x, coords, wq, wk, wv, rpe_w, alpha, ln1_s, ln1_b)


NSORT = 2 * G
NWORK = 32
CHUNK = N // 16


def _sc_argsort(qh2, kh2, shifts_f):
    mesh = plsc.VectorSubcoreMesh(core_axis_name="c", subcore_axis_name="s")
    import dataclasses
    cp = pltpu.CompilerParams()
    if "needs_layout_passes" in pltpu.CompilerParams.__dataclass_fields__:
        cp = dataclasses.replace(cp, needs_layout_passes=False)

    @functools.partial(
        pl.kernel, mesh=mesh,
        out_type=[
            jax.ShapeDtypeStruct((G, N), jnp.int32),
            jax.ShapeDtypeStruct((G, N), jnp.int32),
            jax.ShapeDtypeStruct((G, N), jnp.int32),
        ],
        scratch_types=[
            pltpu.VMEM((N,), jnp.int32),
            pltpu.VMEM((N,), jnp.int32),
            pltpu.VMEM((N,), jnp.int32),
            pltpu.VMEM((N,), jnp.int32),
            pltpu.VMEM((N,), jnp.float32),
            pltpu.VMEM((N,), jnp.float32),
            pltpu.VMEM((N,), jnp.float32),
            pltpu.VMEM((4096,), jnp.int32),
            pltpu.VMEM((16,), jnp.float32),
            pltpu.VMEM((16,), jnp.float32),
            pltpu.SMEM((4,), jnp.int32),
        ],
        compiler_params=cp,
    )
    def kern(qh_hbm, kh_hbm, sh_hbm, qi_hbm, ki_hbm, si_hbm,
             k0, k1, v0, v1, hp, ho, shv, hist, mxv, mnv, run):
        wid = lax.axis_index("s") * 2 + lax.axis_index("c")
        lane = lax.broadcasted_iota(jnp.int32, (16,), 0)
        lane_chunk = lane * CHUNK
        ones = jnp.ones((16,), jnp.int32)
        zeros = jnp.zeros((16,), jnp.int32)
        pltpu.sync_copy(sh_hbm, shv)

        for a0 in range(2):
            a = wid + NWORK * a0

            @pl.when(a < NSORT)
            def _():
                is_q = a < G
                am = lax.rem(a, G)
                h = am // NH
                rowoff = h * N
                nhoff = lax.rem(am, NH) * (H * N)

                @pl.when(is_q)
                def _():
                    pltpu.sync_copy(qh_hbm.at[am], hp)
                    pltpu.sync_copy(kh_hbm.at[am], ho)

                @pl.when(jnp.logical_not(is_q))
                def _():
                    pltpu.sync_copy(kh_hbm.at[am], hp)
                    pltpu.sync_copy(qh_hbm.at[am], ho)

                mxv[...] = jnp.full((16,), -jnp.inf, jnp.float32)
                mnv[...] = jnp.full((16,), jnp.inf, jnp.float32)

                @pl.loop(0, CHUNK)
                def _(i):
                    s = pl.ds(i * 16, 16)
                    pv = hp[s]
                    ov = ho[s]
                    mxv[...] = jnp.maximum(mxv[...], jnp.maximum(pv, ov))
                    mnv[...] = jnp.minimum(mnv[...], jnp.minimum(pv, ov))

                lo = jnp.min(mnv[...])
                rng = jnp.max(mxv[...]) - lo
                rng_v = jnp.zeros((16,), jnp.float32) + rng
                scale = jnp.where(rng_v > 0.0, 16777215.0 / (16.0 * rng_v), 0.0)

                @pl.loop(0, CHUNK)
                def _(i):
                    s = pl.ds(i * 16, 16)
                    kv = hp[s] + shv[s] * rng
                    k0[s] = jnp.minimum((kv - lo) * scale, 16777215.0).astype(jnp.int32)
                    v0[s] = lane + (i * 16)

                for p in range(3):
                    src_k, src_v = (k0, v0) if p % 2 == 0 else (k1, v1)
                    dst_k, dst_v = (k1, v1) if p % 2 == 0 else (k0, v0)
                    shift = 8 * p

                    @pl.loop(0, 256)
                    def _(b):
                        hist[pl.ds(b * 16, 16)] = zeros

                    @pl.loop(0, CHUNK)
                    def _(i):
                        idx = lane_chunk + i
                        kv = plsc.load_gather(src_k, [idx])
                        digit = lax.shift_right_logical(kv, shift) & 255
                        plsc.addupdate_scatter(hist, [digit * 16 + lane], ones)

                    run[0] = 0

                    @pl.loop(0, 256)
                    def _(b):
                        h_b = hist[pl.ds(b * 16, 16)]
                        excl = plsc.cumsum(h_b) - h_b
                        r = run[0]
                        hist[pl.ds(b * 16, 16)] = excl + r
                        run[0] = r + jnp.sum(h_b)

                    @pl.loop(0, CHUNK)
                    def _(i):
                        idx = lane_chunk + i
                        kv = plsc.load_gather(src_k, [idx])
                        vv = plsc.load_gather(src_v, [idx])
                        digit = lax.shift_right_logical(kv, shift) & 255
                        dl = digit * 16 + lane
                        cur = plsc.load_gather(hist, [dl])
                        plsc.store_scatter(dst_k, [cur], kv)
                        plsc.store_scatter(dst_v, [cur], vv)
                        plsc.addupdate_scatter(hist, [dl], ones)

                @pl.loop(0, CHUNK)
                def _(i):
                    s = pl.ds(i * 16, 16)
                    v1[s] = v1[s] + rowoff

                @pl.when(is_q)
                def _():
                    pltpu.sync_copy(v1, qi_hbm.at[am])

                    @pl.loop(0, CHUNK)
                    def _(i):
                        s = pl.ds(i * 16, 16)
                        k0[s] = v1[s] + nhoff

                    pltpu.sync_copy(k0, si_hbm.at[am])

                @pl.when(jnp.logical_not(is_q))
                def _():
                    pltpu.sync_copy(v1, ki_hbm.at[am])

    return kern(qh2, kh2, shifts_f)


def _sc_gather(q_tab, k_tab, v_tab, q_idx, k_idx):
    mesh = plsc.VectorSubcoreMesh(core_axis_name="c", subcore_axis_name="s")

    @functools.partial(
        pl.kernel, mesh=mesh,
        out_type=[
            jax.ShapeDtypeStruct((NIDX, HP), jnp.float32),
            jax.ShapeDtypeStruct((NIDX, HP), jnp.float32),
            jax.ShapeDtypeStruct((NIDX, HP), jnp.float32),
        ],
    )
    def kern(qt_hbm, kt_hbm, vt_hbm, qi_hbm, ki_hbm, oq_hbm, ok_hbm, ov_hbm):
        def body(qi_v, ki_v, oq_v, ok_v, ov_v):
            def inner(s1, s2, s3):
                c1 = pltpu.async_copy(qt_hbm.at[qi_v.at[0]], oq_v, s1)
                c2 = pltpu.async_copy(kt_hbm.at[ki_v.at[0]], ok_v, s2)
                c3 = pltpu.async_copy(vt_hbm.at[ki_v.at[0]], ov_v, s3)
                c1.wait()
                c2.wait()
                c3.wait()

            pl.run_scoped(inner, pltpu.SemaphoreType.DMA,
                          pltpu.SemaphoreType.DMA, pltpu.SemaphoreType.DMA)

        pw = N // W_SC
        pltpu.emit_pipeline(
            body,
            grid=(NIDX // W_SC,),
            in_specs=[
                pl.BlockSpec((1, W_SC), lambda i: (i // pw, i % pw)),
                pl.BlockSpec((1, W_SC), lambda i: (i // pw, i % pw)),
            ],
            out_specs=[
                pl.BlockSpec((W_SC, HP), lambda i: (i, 0)),
                pl.BlockSpec((W_SC, HP), lambda i: (i, 0)),
                pl.BlockSpec((W_SC, HP), lambda i: (i, 0)),
            ],
            core_axis_name=("c", "s"),
            dimension_semantics=(pltpu.PARALLEL,),
        )(qi_hbm, ki_hbm, oq_hbm, ok_hbm, ov_hbm)

    return kern(q_tab, k_tab, v_tab, q_idx, k_idx)


def _sc_scatter(rows, idx):
    mesh = plsc.VectorSubcoreMesh(core_axis_name="c", subcore_axis_name="s")

    @functools.partial(
        pl.kernel, mesh=mesh,
        out_type=jax.ShapeDtypeStruct((NH * H * N, HP), jnp.float32),
    )
    def kern(rows_hbm, idx_hbm, out_hbm):
        def body(rows_v, idx_v):
            pltpu.sync_copy(rows_v, out_hbm.at[idx_v.at[0]])

        pw = N // W_SC
        pltpu.emit_pipeline(
            body,
            grid=(NIDX // W_SC,),
            in_specs=[
                pl.BlockSpec((W_SC, HP), lambda i: (i, 0)),
                pl.BlockSpec((1, W_SC), lambda i: (i // pw, i % pw)),
            ],
            out_specs=[],
            core_axis_name=("c", "s"),
            dimension_semantics=(pltpu.PARALLEL,),
        )(rows_hbm, idx_hbm)

    return kern(rows, idx)


def _attn_body(sq_ref, sk_ref, sv_ref, out_ref):
    nb = B_ATTN // BS
    excol = (lax.broadcasted_iota(jnp.int32, (BS, HP), 1) == D
             ).astype(jnp.float32)
    for b in range(nb):
        sq = sq_ref[0, b * BS:(b + 1) * BS, :]
        sk = sk_ref[0, b * BS:(b + 1) * BS, :]
        sv = sv_ref[0, b * BS:(b + 1) * BS, :]
        qsq = -0.5 * jnp.sum(sq * sq, axis=-1, keepdims=True)
        ksq = -0.5 * jnp.sum(sk * sk, axis=-1)
        scores = lax.dot_general(sq, sk, (((1,), (1,)), ((), ())),
                                 preferred_element_type=jnp.float32)
        dists = jnp.exp(jnp.minimum(scores + qsq + ksq[None, :], 0.0))
        o = lax.dot_general(dists, sv + excol, (((1,), (0,)), ((), ())),
                            preferred_element_type=jnp.float32)
        out_ref[0, b * BS:(b + 1) * BS, :] = o + 1e-20 * excol


def _attn_call(sq, sk, sv):
    nb = N // B_ATTN
    return pl.pallas_call(
        _attn_body,
        grid=(G, nb),
        in_specs=[
            pl.BlockSpec((1, B_ATTN, HP), lambda g, i: (g, i, 0)),
            pl.BlockSpec((1, B_ATTN, HP), lambda g, i: (g, i, 0)),
            pl.BlockSpec((1, B_ATTN, HP), lambda g, i: (g, i, 0)),
        ],
        out_specs=pl.BlockSpec((1, B_ATTN, HP), lambda g, i: (g, i, 0)),
        out_shape=jax.ShapeDtypeStruct((G, N, HP), jnp.float32),
    )(sq, sk, sv)


def _final_body(so_ref, x_ref, outw_ref, outb_ref, ln2s_ref, ln2b_ref,
                ff1w_ref, ff1b_ref, ff2w_ref, ff2b_ref, y_ref):
    s = so_ref[...]
    s = s[0] + s[1] + s[2]
    o = s[..., :D]
    den = s[..., D:D + 1]
    outh = o / den
    dn = (((1,), (1,)), ((), ()))
    aggr = outb_ref[...][None, :]
    for h in range(H):
        w_h = outw_ref[:, h * D:(h + 1) * D]
        aggr = aggr + lax.dot_general(outh[h], w_h, dn,
                                      preferred_element_type=jnp.float32)
    x1 = x_ref[...] + aggr
    m = jnp.mean(x1, axis=-1, keepdims=True)
    v = jnp.mean((x1 - m) ** 2, axis=-1, keepdims=True)
    xn2 = (x1 - m) / jnp.sqrt(v + 1e-5) * ln2s_ref[...][None, :] + ln2b_ref[...][None, :]
    h1 = lax.dot_general(xn2, ff1w_ref[...], dn,
                         preferred_element_type=jnp.float32) + ff1b_ref[...][None, :]
    h1 = h1 * jax.nn.sigmoid(h1)
    h2 = lax.dot_general(h1, ff2w_ref[...], dn,
                         preferred_element_type=jnp.float32) + ff2b_ref[...][None, :]
    y_ref[...] = x1 + h2


def _final_call(so, x, out_w, out_b, ln2_s, ln2_b, ff1_w, ff1_b, ff2_w, ff2_b):
    nb = N // B_FIN
    full = lambda shp: pl.BlockSpec(shp, lambda i: tuple(0 for _ in shp))
    return pl.pallas_call(
        _final_body,
        grid=(nb,),
        in_specs=[
            pl.BlockSpec((NH, H, B_FIN, HP), lambda i: (0, 0, i, 0)),
            pl.BlockSpec((B_FIN, D), lambda i: (i, 0)),
            full((D, H * D)), full((D,)), full((D,)), full((D,)),
            full((D, D)), full((D,)), full((D, D)), full((D,)),
        ],
        out_specs=pl.BlockSpec((B_FIN, D), lambda i: (i, 0)),
        out_shape=jax.ShapeDtypeStruct((N, D), jnp.float32),
    )(so, x, out_w, out_b, ln2_s, ln2_b, ff1_w, ff1_b, ff2_w, ff2_b)


def kernel(x, coords, combined_shifts, wq, wk, wv, out_w, out_b, rpe_w,
           ln1_s, ln1_b, ln2_s, ln2_b, ff1_w, ff1_b, ff2_w, ff2_b, alpha):
    q_tab, k_tab, v_tab, qh, kh = _prep_call(
        x, coords, wq, wk, wv, rpe_w, alpha, ln1_s, ln1_b)

    q_flat, k_flat, s_idx = _sc_argsort(
        qh.reshape(G, N), kh.reshape(G, N),
        combined_shifts.astype(jnp.float32))

    sq, sk, sv = _sc_gather(q_tab.reshape(H * N, HP), k_tab.reshape(H * N, HP),
                            v_tab.reshape(H * N, HP), q_flat, k_flat)

    so = _attn_call(sq.reshape(G, N, HP), sk.reshape(G, N, HP),
                    sv.reshape(G, N, HP))

    uns = _sc_scatter(so.reshape(NIDX, HP), s_idx)

    return _final_call(uns.reshape(NH, H, N, HP), x, out_w, out_b,
                       ln2_s, ln2_b, ff1_w, ff1_b, ff2_w, ff2_b)
